# Initial kernel scaffold; baseline (speedup 1.0000x reference)
import jax
import jax.numpy as jnp
from jax.experimental import pallas as pl


def kernel(x, table):
    raise NotImplementedError("write your pallas kernel here")


# speedup vs baseline: 6.4805x; 6.4805x over previous
import jax
import jax.numpy as jnp
from jax.experimental import pallas as pl


def _body(table_ref, out_ref):
    ts = table_ref.shape[0]
    b = out_ref.shape[1]
    d = table_ref.shape[1]
    out_ref[...] = jnp.broadcast_to(table_ref[...][:, None, :], (ts, b, d))


def kernel(x, table):
    s = x.shape[0]
    b = x.shape[1]
    d = table.shape[1]
    ts = 256
    grid = (s // ts,)
    return pl.pallas_call(
        _body,
        grid=grid,
        in_specs=[pl.BlockSpec((ts, d), lambda i: (i, 0))],
        out_specs=pl.BlockSpec((ts, b, d), lambda i: (i, 0, 0)),
        out_shape=jax.ShapeDtypeStruct((s, b, d), jnp.float32),
    )(table)
